# Initial kernel scaffold; baseline (speedup 1.0000x reference)
import jax
import jax.numpy as jnp
from jax.experimental import pallas as pl


def kernel(x0, edge_index0, x1, edge_index1, clusters0, params):
    raise NotImplementedError("write your pallas kernel here")


# speedup vs baseline: 3.1109x; 3.1109x over previous
import functools

import jax
import jax.numpy as jnp
from jax import lax
from jax.experimental import pallas as pl
from jax.experimental.pallas import tpu as pltpu
from jax.experimental.pallas import tpu_sc as plsc

F32 = jnp.float32
I32 = jnp.int32

N0, E0, N1, E1 = 10000, 320000, 2500, 80000
NC, NS, LANES = 2, 16, 16
NW = NC * NS

E0P = 327680
E1P = 81920
SN0 = 10240
SN1 = 2560
NT0 = N0 + 8
NT1 = N1 + 8
N0P = 10240

_MESH = plsc.VectorSubcoreMesh(core_axis_name="c", subcore_axis_name="s")
_NOTC = pltpu.CompilerParams(use_tc_tiling_on_sc=False)
_NOLAYOUT = pltpu.CompilerParams(needs_layout_passes=False)


def _take16(v, idx):
    return v.at[idx].get(mode="promise_in_bounds")


def _wid():
    return lax.axis_index("s") * NC + lax.axis_index("c")


def _make_gather(NT, C, M, GC):
    npt = M // NW
    nch = npt // GC

    @functools.partial(
        pl.kernel,
        out_type=jax.ShapeDtypeStruct((M, C), F32),
        mesh=_MESH,
        compiler_params=_NOTC,
        scratch_types=[
            pltpu.VMEM((GC,), I32),
            pltpu.VMEM((GC, C), F32),
            pltpu.SemaphoreType.DMA,
        ],
    )
    def k(table_hbm, idx_hbm, out_hbm, idx_v, rows_v, sem):
        base = _wid() * npt

        def body(j, carry):
            off = base + j * GC
            pltpu.sync_copy(idx_hbm.at[pl.ds(off, GC)], idx_v)
            pltpu.async_copy(table_hbm.at[idx_v], rows_v, sem).wait()
            pltpu.sync_copy(rows_v, out_hbm.at[pl.ds(off, GC)])
            return carry

        lax.fori_loop(0, nch, body, 0)

    return k


def _make_scatter_max(SN, EP, CH=512):
    npt = EP // NW
    nch = npt // CH
    TW = SN * 8

    @functools.partial(
        pl.kernel,
        out_type=jax.ShapeDtypeStruct((NW * TW,), F32),
        mesh=_MESH,
        compiler_params=_NOLAYOUT,
        scratch_types=[
            pltpu.VMEM((TW,), F32),
            pltpu.VMEM((8, CH), F32),
            pltpu.VMEM((CH,), I32),
            pltpu.SemaphoreType.DMA,
        ],
    )
    def k(m_hbm, dst_hbm, out_hbm, tbl, m_v, idx_v, sem):
        wid = _wid()
        base = wid * npt
        neginf = jnp.full((LANES,), -jnp.inf, F32)

        def ini(i, carry):
            tbl[pl.ds(i * LANES, LANES)] = neginf
            return carry

        lax.fori_loop(0, TW // LANES, ini, 0)

        lane = lax.iota(I32, LANES)
        lane7 = lane & 7
        rot8 = lane ^ 8
        half = lane < 8

        def chunk(ch, carry):
            off = base + ch * CH
            pltpu.sync_copy(dst_hbm.at[pl.ds(off, CH)], idx_v)
            pltpu.sync_copy(m_hbm.at[:, pl.ds(off, CH)], m_v)

            def group(g, c2):
                dstv = idx_v[pl.ds(g * LANES, LANES)]
                for p in range(8):
                    patt = (lane >> 3) + 2 * p
                    dv = _take16(dstv, patt)
                    idxv = dv * 8 + lane7
                    cur = plsc.load_gather(tbl, [idxv])
                    colv = (lane >> 3) + (g * LANES + 2 * p)
                    mrow = plsc.load_gather(m_v, [lane7, colv])
                    mx = jnp.maximum(cur, mrow)
                    dvr = _take16(dv, rot8)
                    dup = dv == dvr
                    mxr = _take16(mx, rot8)
                    val = jnp.where(dup, jnp.maximum(mx, mxr), mx)
                    wm = jnp.logical_or(jnp.logical_not(dup), half)
                    plsc.store_scatter(tbl, [idxv], val, mask=wm)
                return c2

            lax.fori_loop(0, CH // LANES, group, 0)
            return carry

        lax.fori_loop(0, nch, chunk, 0)
        pltpu.sync_copy(tbl, out_hbm.at[pl.ds(wid * TW, TW)])

    return k


def _make_pool(SNt, NP, CH):
    npt = NP // NW
    nch = npt // CH
    TW = SNt * 16

    @functools.partial(
        pl.kernel,
        out_type=jax.ShapeDtypeStruct((NW * TW,), F32),
        mesh=_MESH,
        compiler_params=_NOLAYOUT,
        scratch_types=[
            pltpu.VMEM((TW,), F32),
            pltpu.VMEM((CH * 16,), F32),
            pltpu.VMEM((CH,), I32),
            pltpu.SemaphoreType.DMA,
        ],
    )
    def k(val_hbm, idx_hbm, out_hbm, tbl, v_v, idx_v, sem):
        wid = _wid()
        base = wid * npt
        zero = jnp.zeros((LANES,), F32)

        def ini(i, carry):
            tbl[pl.ds(i * LANES, LANES)] = zero
            return carry

        lax.fori_loop(0, TW // LANES, ini, 0)
        lane = lax.iota(I32, LANES)

        def chunk(ch, carry):
            off = base + ch * CH
            pltpu.sync_copy(idx_hbm.at[pl.ds(off, CH)], idx_v)
            pltpu.sync_copy(val_hbm.at[pl.ds(off * 16, CH * 16)], v_v)

            def group(g, c2):
                cv = idx_v[pl.ds(g * LANES, LANES)]
                for q in range(LANES):
                    cq = _take16(cv, jnp.full((LANES,), q, I32))
                    idxv = cq * 16 + lane
                    cur = plsc.load_gather(tbl, [idxv])
                    val = v_v[pl.ds((g * LANES + q) * 16, LANES)]
                    plsc.store_scatter(tbl, [idxv], cur + val)
                return c2

            lax.fori_loop(0, CH // LANES, group, 0)
            return carry

        lax.fori_loop(0, nch, chunk, 0)
        pltpu.sync_copy(tbl, out_hbm.at[pl.ds(wid * TW, TW)])

    return k


def _make_edge_mlp(C, EP, BEp=512):
    K = 128 // (2 * C)
    MP = 2 * EP * C // 128
    KE = K * BEp

    def body(xp_ref, we_ref, b1_ref, w2_ref, b2_ref, w3_ref, b3_ref, o_ref):
        xp = xp_ref[...]
        shifted = jnp.concatenate(
            [jnp.zeros((BEp, C), F32), xp[:, :128 - C]], axis=1)
        lanes = lax.broadcasted_iota(I32, (BEp, 128), 1)
        odd = (lanes // C) % 2 == 1
        xd = xp - jnp.where(odd, shifted, 0.0)
        hp = jnp.dot(xd, we_ref[...], preferred_element_type=F32)
        h = hp.reshape(KE, 128)
        h = jnp.maximum(h + b1_ref[...], 0.0)
        h = jnp.maximum(jnp.dot(h, w2_ref[...], preferred_element_type=F32)
                        + b2_ref[...], 0.0)
        m = jnp.dot(h, w3_ref[...], preferred_element_type=F32) + b3_ref[...]
        o_ref[...] = m.T

    return pl.pallas_call(
        body,
        grid=(MP // BEp,),
        in_specs=[
            pl.BlockSpec((BEp, 128), lambda i: (i, 0)),
            pl.BlockSpec((128, K * 128), lambda i: (0, 0)),
            pl.BlockSpec((1, 128), lambda i: (0, 0)),
            pl.BlockSpec((128, 128), lambda i: (0, 0)),
            pl.BlockSpec((1, 128), lambda i: (0, 0)),
            pl.BlockSpec((128, 8), lambda i: (0, 0)),
            pl.BlockSpec((1, 8), lambda i: (0, 0)),
        ],
        out_specs=pl.BlockSpec((8, KE), lambda i: (0, i)),
        out_shape=jax.ShapeDtypeStruct((8, EP), F32),
    )


def _make_reduce(SN, NV, CHR):
    ROWS = SN * 8 // 128
    VW = NV * 8
    FR, REM = VW // 128, VW % 128

    def body(p_ref, y_ref, st_ref):
        i = pl.program_id(0)
        y = jnp.max(p_ref[...], axis=0)
        y = jnp.where(jnp.isneginf(y), 0.0, y)
        y = jnp.maximum(y, 0.0)
        y_ref[...] = y
        rows = i * CHR + lax.broadcasted_iota(I32, (CHR, 128), 0)
        valid = rows < FR
        if REM:
            lanes = lax.broadcasted_iota(I32, (CHR, 128), 1)
            valid = valid | ((rows == FR) & (lanes < REM))
        ym = jnp.where(valid, y, 0.0)
        s = jnp.sum(ym, axis=0, keepdims=True)
        ss = jnp.sum(ym * ym, axis=0, keepdims=True)

        @pl.when(i == 0)
        def _():
            st_ref[...] = jnp.zeros_like(st_ref)

        st_ref[...] += jnp.concatenate([s, ss], axis=0)

    return pl.pallas_call(
        body,
        grid=(ROWS // CHR,),
        in_specs=[pl.BlockSpec((NW, CHR, 128), lambda i: (0, i, 0))],
        out_specs=[
            pl.BlockSpec((CHR, 128), lambda i: (i, 0)),
            pl.BlockSpec((2, 128), lambda i: (0, 0)),
        ],
        out_shape=[
            jax.ShapeDtypeStruct((ROWS, 128), F32),
            jax.ShapeDtypeStruct((2, 128), F32),
        ],
    )


def _make_bn_packed(ROWS, NV):
    def body(y_ref, st_ref, g_ref, b_ref, o_ref):
        mean = st_ref[0:1, :] / NV
        var = st_ref[1:2, :] / NV - mean * mean
        o_ref[...] = (g_ref[...] * (y_ref[...] - mean)
                      * lax.rsqrt(var + 1e-5) + b_ref[...])

    return pl.pallas_call(
        body,
        in_specs=[
            pl.BlockSpec((ROWS, 128), lambda: (0, 0)),
            pl.BlockSpec((2, 128), lambda: (0, 0)),
            pl.BlockSpec((1, 128), lambda: (0, 0)),
            pl.BlockSpec((1, 128), lambda: (0, 0)),
        ],
        out_specs=pl.BlockSpec((ROWS, 128), lambda: (0, 0)),
        out_shape=jax.ShapeDtypeStruct((ROWS, 128), F32),
    )


def _make_pool_sum(ROWS):
    def body(p_ref, o_ref):
        o_ref[...] = jnp.sum(p_ref[...], axis=0)

    return pl.pallas_call(
        body,
        in_specs=[pl.BlockSpec((NW, ROWS, 128), lambda: (0, 0, 0))],
        out_specs=pl.BlockSpec((ROWS, 128), lambda: (0, 0)),
        out_shape=jax.ShapeDtypeStruct((ROWS, 128), F32),
    )


def _make_pool_div(NR):
    def body(s_ref, o_ref):
        o_ref[...] = s_ref[:, 0:8] / jnp.maximum(s_ref[:, 8:9], 1.0)

    return pl.pallas_call(
        body,
        in_specs=[pl.BlockSpec((NR, 16), lambda: (0, 0))],
        out_specs=pl.BlockSpec((NR, 8), lambda: (0, 0)),
        out_shape=jax.ShapeDtypeStruct((NR, 8), F32),
    )


def _make_final(NV, BE=2000):
    def body(y_ref, st_ref, g_ref, b_ref, w1_ref, c1_ref, w2_ref, c2_ref,
             w3_ref, c3_ref, o_ref):
        mean = st_ref[0:1, :] / NV
        var = st_ref[1:2, :] / NV - mean * mean
        x = (g_ref[...] * (y_ref[...] - mean) * lax.rsqrt(var + 1e-5)
             + b_ref[...])
        h = jnp.maximum(jnp.dot(x, w1_ref[...], preferred_element_type=F32)
                        + c1_ref[...], 0.0)
        h = jnp.maximum(jnp.dot(h, w2_ref[...], preferred_element_type=F32)
                        + c2_ref[...], 0.0)
        o_ref[...] = (jnp.dot(h, w3_ref[...], preferred_element_type=F32)
                      + c3_ref[...])

    return pl.pallas_call(
        body,
        grid=(NV // BE,),
        in_specs=[
            pl.BlockSpec((BE, 8), lambda i: (i, 0)),
            pl.BlockSpec((2, 8), lambda i: (0, 0)),
            pl.BlockSpec((1, 8), lambda i: (0, 0)),
            pl.BlockSpec((1, 8), lambda i: (0, 0)),
            pl.BlockSpec((8, 128), lambda i: (0, 0)),
            pl.BlockSpec((1, 128), lambda i: (0, 0)),
            pl.BlockSpec((128, 128), lambda i: (0, 0)),
            pl.BlockSpec((1, 128), lambda i: (0, 0)),
            pl.BlockSpec((128, 1), lambda i: (0, 0)),
            pl.BlockSpec((1, 1), lambda i: (0, 0)),
        ],
        out_specs=pl.BlockSpec((BE, 1), lambda i: (i, 0)),
        out_shape=jax.ShapeDtypeStruct((NV, 1), F32),
    )


_g_conv0 = _make_gather(NT0, 16, 2 * E0P, 128)
_g_conv1 = _make_gather(NT1, 16, 2 * E1P, 128)
_g_convR1 = _make_gather(NT1, 32, 2 * E1P, 128)
_g_convR0 = _make_gather(NT0, 32, 2 * E0P, 128)
_g_unpool = _make_gather(NT1, 16, N0P, 64)
_s_max0 = _make_scatter_max(SN0, E0P)
_s_max1 = _make_scatter_max(SN1, E1P)
_s_pool = _make_pool(SN1, N0P, N0P // NW)
_mlp0 = _make_edge_mlp(16, E0P)
_mlp1 = _make_edge_mlp(16, E1P)
_mlpR1 = _make_edge_mlp(32, E1P)
_mlpR0 = _make_edge_mlp(32, E0P)
_red0 = _make_reduce(SN0, N0, 160)
_red1 = _make_reduce(SN1, N1, 160)
_bnp0 = _make_bn_packed(SN0 * 8 // 128, N0)
_bnp1 = _make_bn_packed(SN1 * 8 // 128, N1)
_pool_sum = _make_pool_sum(SN1 * 16 // 128)
_pool_div = _make_pool_div(2504)
_final = _make_final(N0)


def _pad_rows(a, rows):
    return jnp.concatenate(
        [a, jnp.zeros((rows - a.shape[0], a.shape[1]), a.dtype)], axis=0)


def _pad_cols(a, cols):
    return jnp.concatenate(
        [a, jnp.zeros((a.shape[0], cols - a.shape[1]), a.dtype)], axis=1)


def _pad_idx(a, n, fill):
    return jnp.concatenate(
        [a, jnp.full((n - a.shape[0],), fill, a.dtype)], axis=0)


def _expand_w(layer0, d, C):
    W, b = layer0
    K = 128 // (2 * C)
    wa = W[:d]
    wb = W[d:]
    we = jnp.zeros((128, K * 128), F32)
    for k in range(K):
        we = we.at[2 * C * k:2 * C * k + d, 128 * k:128 * (k + 1)].set(wa)
        we = we.at[2 * C * k + C:2 * C * k + C + d,
                   128 * k:128 * (k + 1)].set(wb)
    return we, b.reshape(1, 128)


def _tile128(v8):
    return jnp.tile(v8.reshape(1, 8), (1, 16))


def _conv(table, gidx, dstp, layers, d, C, EP, SN, gather, mlp, smax, red):
    we, b1 = _expand_w(layers[0], d, C)
    w2, b2 = layers[1]
    w3, b3 = layers[2]
    g = gather(table, gidx)
    xp = g.reshape(2 * EP * C // 128, 128)
    mT = mlp(xp, we, b1, w2, b2.reshape(1, 128), w3, b3.reshape(1, 8))
    parts = smax(mT, dstp)
    y, st = red(parts.reshape(NW, SN * 8 // 128, 128))
    return y, st


def kernel(x0, edge_index0, x1, edge_index1, clusters0, params):
    edge_index0 = edge_index0.astype(I32)
    edge_index1 = edge_index1.astype(I32)
    clusters0 = clusters0.astype(I32)

    src0 = _pad_idx(edge_index0[0], E0P, N0)
    dst0 = _pad_idx(edge_index0[1], E0P, N0)
    gidx0 = jnp.stack([dst0, src0], axis=1).reshape(-1)
    src1 = _pad_idx(edge_index1[0], E1P, N1)
    dst1 = _pad_idx(edge_index1[1], E1P, N1)
    gidx1 = jnp.stack([dst1, src1], axis=1).reshape(-1)

    t0 = _pad_rows(_pad_cols(x0, 16), NT0)
    y0, st0p = _conv(t0, gidx0, dst0, params["Lconv0"], 3, 16, E0P, SN0,
                     _g_conv0, _mlp0, _s_max0, _red0)
    st0 = st0p.reshape(2, 16, 8).sum(axis=1)
    h0p = _bnp0(y0, jnp.tile(st0, (1, 16)),
                _tile128(params["Lnorm0"][0]), _tile128(params["Lnorm0"][1]))
    h0 = h0p.reshape(SN0, 8)[:N0]

    h0e = jnp.concatenate(
        [h0, jnp.ones((N0, 1), F32), jnp.zeros((N0, 7), F32)], axis=1)
    h0e = _pad_rows(h0e, N0P)
    clp = _pad_idx(clusters0, N0P, N1)
    pp = _s_pool(h0e.reshape(-1), clp)
    sums = _pool_sum(pp.reshape(NW, SN1 * 16 // 128, 128))
    p1 = _pool_div(sums.reshape(SN1, 16)[:2504])[:N1]

    x1f = x1[:, :2]
    t1 = _pad_rows(_pad_cols(jnp.concatenate([x1f, p1], axis=1), 16), NT1)
    y1, st1p = _conv(t1, gidx1, dst1, params["Lconv1"], 10, 16, E1P, SN1,
                     _g_conv1, _mlp1, _s_max1, _red1)
    st1 = st1p.reshape(2, 16, 8).sum(axis=1)
    X1p = _bnp1(y1, jnp.tile(st1, (1, 16)),
                _tile128(params["Lnorm1"][0]), _tile128(params["Lnorm1"][1]))
    X1 = X1p.reshape(SN1, 8)[:N1]

    t2 = _pad_rows(_pad_cols(jnp.concatenate([x1f, p1, X1], axis=1), 32), NT1)
    y2, st2p = _conv(t2, gidx1, dst1, params["Rconv1"], 18, 32, E1P, SN1,
                     _g_convR1, _mlpR1, _s_max1, _red1)
    st2 = st2p.reshape(2, 16, 8).sum(axis=1)
    X2p = _bnp1(y2, jnp.tile(st2, (1, 16)),
                _tile128(params["Rnorm1"][0]), _tile128(params["Rnorm1"][1]))
    X2 = X2p.reshape(SN1, 8)[:N1]

    x2t = _pad_rows(_pad_cols(X2, 16), NT1)
    clg = _pad_idx(clusters0, N0P, 0)
    X3 = _g_unpool(x2t, clg)[:N0, :8]

    t3 = _pad_rows(_pad_cols(
        jnp.concatenate([x0[:, :2], h0, X3], axis=1), 32), NT0)
    y3, st3p = _conv(t3, gidx0, dst0, params["Rconv0"], 18, 32, E0P, SN0,
                     _g_convR0, _mlpR0, _s_max0, _red0)
    st3 = st3p.reshape(2, 16, 8).sum(axis=1)

    (w1, c1), (w2, c2), (w3, c3) = params["mlp_out"]
    out = _final(y3.reshape(SN0, 8)[:N0], st3,
                 params["Rnorm0"][0].reshape(1, 8),
                 params["Rnorm0"][1].reshape(1, 8),
                 w1, c1.reshape(1, 128), w2, c2.reshape(1, 128),
                 w3, c3.reshape(1, 1))
    return out


# speedup vs baseline: 3.7055x; 1.1911x over previous
import functools

import jax
import jax.numpy as jnp
from jax import lax
from jax.experimental import pallas as pl
from jax.experimental.pallas import tpu as pltpu
from jax.experimental.pallas import tpu_sc as plsc

F32 = jnp.float32
I32 = jnp.int32

N0, E0, N1, E1 = 10000, 320000, 2500, 80000
NC, NS, LANES = 2, 16, 16
NW = NC * NS

E0P = 327680
E1P = 81920
SN0 = 10240
SN1 = 2560
NT0 = N0 + 8
NT1 = N1 + 8
N0P = 10240

_MESH = plsc.VectorSubcoreMesh(core_axis_name="c", subcore_axis_name="s")
_NOTC = pltpu.CompilerParams(use_tc_tiling_on_sc=False)
_NOLAYOUT = pltpu.CompilerParams(needs_layout_passes=False)


def _take16(v, idx):
    return v.at[idx].get(mode="promise_in_bounds")


def _wid():
    return lax.axis_index("s") * NC + lax.axis_index("c")


def _make_gather(NT, C, M, GC, NB=4):
    npt = M // NW
    nch = npt // GC
    nrounds = nch // NB

    @functools.partial(
        pl.kernel,
        out_type=jax.ShapeDtypeStruct((M, C), F32),
        mesh=_MESH,
        compiler_params=_NOTC,
        scratch_types=[
            pltpu.VMEM((npt,), I32),
            pltpu.VMEM((NB, GC, C), F32),
            pltpu.SemaphoreType.DMA,
            pltpu.SemaphoreType.DMA,
        ],
    )
    def k(table_hbm, idx_hbm, out_hbm, idx_v, rows_v, sem_g, sem_o):
        base = _wid() * npt
        pltpu.sync_copy(idx_hbm.at[pl.ds(base, npt)], idx_v)

        def rnd(r, carry):
            gats = []
            for b in range(NB):
                j = r * NB + b
                gats.append(pltpu.async_copy(
                    table_hbm.at[idx_v.at[pl.ds(j * GC, GC)]],
                    rows_v.at[b], sem_g))
            outs = []
            for b in range(NB):
                j = r * NB + b
                gats[b].wait()
                outs.append(pltpu.async_copy(
                    rows_v.at[b], out_hbm.at[pl.ds(base + j * GC, GC)],
                    sem_o))
            for b in range(NB):
                outs[b].wait()
            return carry

        lax.fori_loop(0, nrounds, rnd, 0)

    return k


def _make_scatter_max(SN, EP, CH=512):
    npt = EP // NW
    nch = npt // CH
    TW = SN * 8

    @functools.partial(
        pl.kernel,
        out_type=jax.ShapeDtypeStruct((NW * TW,), F32),
        mesh=_MESH,
        compiler_params=_NOLAYOUT,
        scratch_types=[
            pltpu.VMEM((TW,), F32),
            pltpu.VMEM((8, CH), F32),
            pltpu.VMEM((CH,), I32),
            pltpu.SemaphoreType.DMA,
        ],
    )
    def k(m_hbm, dst_hbm, out_hbm, tbl, m_v, idx_v, sem):
        wid = _wid()
        base = wid * npt
        neginf = jnp.full((LANES,), -jnp.inf, F32)

        def ini(i, carry):
            tbl[pl.ds(i * LANES, LANES)] = neginf
            return carry

        lax.fori_loop(0, TW // LANES, ini, 0)

        lane = lax.iota(I32, LANES)
        lane7 = lane & 7
        rot8 = lane ^ 8
        half = lane < 8

        def chunk(ch, carry):
            off = base + ch * CH
            pltpu.sync_copy(dst_hbm.at[pl.ds(off, CH)], idx_v)
            pltpu.sync_copy(m_hbm.at[:, pl.ds(off, CH)], m_v)

            def group(g, c2):
                dstv = idx_v[pl.ds(g * LANES, LANES)]
                for p in range(8):
                    patt = (lane >> 3) + 2 * p
                    dv = _take16(dstv, patt)
                    idxv = dv * 8 + lane7
                    colv = (lane >> 3) + (g * LANES + 2 * p)
                    mrow = plsc.load_gather(m_v, [lane7, colv])
                    dup = dv == _take16(dv, rot8)
                    pm = jnp.where(dup,
                                   jnp.maximum(mrow, _take16(mrow, rot8)),
                                   mrow)
                    wm = jnp.logical_or(jnp.logical_not(dup), half)
                    cur = plsc.load_gather(tbl, [idxv])
                    plsc.store_scatter(tbl, [idxv], jnp.maximum(cur, pm),
                                       mask=wm)
                return c2

            lax.fori_loop(0, CH // LANES, group, 0)
            return carry

        lax.fori_loop(0, nch, chunk, 0)
        pltpu.sync_copy(tbl, out_hbm.at[pl.ds(wid * TW, TW)])

    return k


def _make_pool(SNt, NP, CH):
    npt = NP // NW
    nch = npt // CH
    TW = SNt * 16

    @functools.partial(
        pl.kernel,
        out_type=jax.ShapeDtypeStruct((NW * TW,), F32),
        mesh=_MESH,
        compiler_params=_NOLAYOUT,
        scratch_types=[
            pltpu.VMEM((TW,), F32),
            pltpu.VMEM((CH * 16,), F32),
            pltpu.VMEM((CH,), I32),
            pltpu.SemaphoreType.DMA,
        ],
    )
    def k(val_hbm, idx_hbm, out_hbm, tbl, v_v, idx_v, sem):
        wid = _wid()
        base = wid * npt
        zero = jnp.zeros((LANES,), F32)

        def ini(i, carry):
            tbl[pl.ds(i * LANES, LANES)] = zero
            return carry

        lax.fori_loop(0, TW // LANES, ini, 0)
        lane = lax.iota(I32, LANES)

        def chunk(ch, carry):
            off = base + ch * CH
            pltpu.sync_copy(idx_hbm.at[pl.ds(off, CH)], idx_v)
            pltpu.sync_copy(val_hbm.at[pl.ds(off * 16, CH * 16)], v_v)

            def group(g, c2):
                cv = idx_v[pl.ds(g * LANES, LANES)]
                for q in range(LANES):
                    cq = _take16(cv, jnp.full((LANES,), q, I32))
                    idxv = cq * 16 + lane
                    cur = plsc.load_gather(tbl, [idxv])
                    val = v_v[pl.ds((g * LANES + q) * 16, LANES)]
                    plsc.store_scatter(tbl, [idxv], cur + val)
                return c2

            lax.fori_loop(0, CH // LANES, group, 0)
            return carry

        lax.fori_loop(0, nch, chunk, 0)
        pltpu.sync_copy(tbl, out_hbm.at[pl.ds(wid * TW, TW)])

    return k


def _make_edge_mlp(C, EP, BEp=512):
    K = 128 // (2 * C)
    MP = 2 * EP * C // 128
    KE = K * BEp

    def body(xp_ref, we_ref, b1_ref, w2_ref, b2_ref, w3_ref, b3_ref, o_ref):
        xp = xp_ref[...]
        shifted = jnp.concatenate(
            [jnp.zeros((BEp, C), F32), xp[:, :128 - C]], axis=1)
        lanes = lax.broadcasted_iota(I32, (BEp, 128), 1)
        odd = (lanes // C) % 2 == 1
        xd = xp - jnp.where(odd, shifted, 0.0)
        hp = jnp.dot(xd, we_ref[...], preferred_element_type=F32)
        h = hp.reshape(KE, 128)
        h = jnp.maximum(h + b1_ref[...], 0.0)
        h = jnp.maximum(jnp.dot(h, w2_ref[...], preferred_element_type=F32)
                        + b2_ref[...], 0.0)
        m = jnp.dot(h, w3_ref[...], preferred_element_type=F32) + b3_ref[...]
        o_ref[...] = m.T

    return pl.pallas_call(
        body,
        grid=(MP // BEp,),
        in_specs=[
            pl.BlockSpec((BEp, 128), lambda i: (i, 0)),
            pl.BlockSpec((128, K * 128), lambda i: (0, 0)),
            pl.BlockSpec((1, 128), lambda i: (0, 0)),
            pl.BlockSpec((128, 128), lambda i: (0, 0)),
            pl.BlockSpec((1, 128), lambda i: (0, 0)),
            pl.BlockSpec((128, 8), lambda i: (0, 0)),
            pl.BlockSpec((1, 8), lambda i: (0, 0)),
        ],
        out_specs=pl.BlockSpec((8, KE), lambda i: (0, i)),
        out_shape=jax.ShapeDtypeStruct((8, EP), F32),
    )


def _make_reduce(SN, NV, CHR):
    ROWS = SN * 8 // 128
    VW = NV * 8
    FR, REM = VW // 128, VW % 128

    def body(p_ref, y_ref, st_ref):
        i = pl.program_id(0)
        y = jnp.max(p_ref[...], axis=0)
        y = jnp.where(jnp.isneginf(y), 0.0, y)
        y = jnp.maximum(y, 0.0)
        y_ref[...] = y
        rows = i * CHR + lax.broadcasted_iota(I32, (CHR, 128), 0)
        valid = rows < FR
        if REM:
            lanes = lax.broadcasted_iota(I32, (CHR, 128), 1)
            valid = valid | ((rows == FR) & (lanes < REM))
        ym = jnp.where(valid, y, 0.0)
        s = jnp.sum(ym, axis=0, keepdims=True)
        ss = jnp.sum(ym * ym, axis=0, keepdims=True)

        @pl.when(i == 0)
        def _():
            st_ref[...] = jnp.zeros_like(st_ref)

        st_ref[...] += jnp.concatenate([s, ss], axis=0)

    return pl.pallas_call(
        body,
        grid=(ROWS // CHR,),
        in_specs=[pl.BlockSpec((NW, CHR, 128), lambda i: (0, i, 0))],
        out_specs=[
            pl.BlockSpec((CHR, 128), lambda i: (i, 0)),
            pl.BlockSpec((2, 128), lambda i: (0, 0)),
        ],
        out_shape=[
            jax.ShapeDtypeStruct((ROWS, 128), F32),
            jax.ShapeDtypeStruct((2, 128), F32),
        ],
    )


def _make_bn_packed(ROWS, NV):
    def body(y_ref, st_ref, g_ref, b_ref, o_ref):
        mean = st_ref[0:1, :] / NV
        var = st_ref[1:2, :] / NV - mean * mean
        o_ref[...] = (g_ref[...] * (y_ref[...] - mean)
                      * lax.rsqrt(var + 1e-5) + b_ref[...])

    return pl.pallas_call(
        body,
        in_specs=[
            pl.BlockSpec((ROWS, 128), lambda: (0, 0)),
            pl.BlockSpec((2, 128), lambda: (0, 0)),
            pl.BlockSpec((1, 128), lambda: (0, 0)),
            pl.BlockSpec((1, 128), lambda: (0, 0)),
        ],
        out_specs=pl.BlockSpec((ROWS, 128), lambda: (0, 0)),
        out_shape=jax.ShapeDtypeStruct((ROWS, 128), F32),
    )


def _make_pool_sum(ROWS):
    def body(p_ref, o_ref):
        o_ref[...] = jnp.sum(p_ref[...], axis=0)

    return pl.pallas_call(
        body,
        in_specs=[pl.BlockSpec((NW, ROWS, 128), lambda: (0, 0, 0))],
        out_specs=pl.BlockSpec((ROWS, 128), lambda: (0, 0)),
        out_shape=jax.ShapeDtypeStruct((ROWS, 128), F32),
    )


def _make_pool_div(NR):
    def body(s_ref, o_ref):
        o_ref[...] = s_ref[:, 0:8] / jnp.maximum(s_ref[:, 8:9], 1.0)

    return pl.pallas_call(
        body,
        in_specs=[pl.BlockSpec((NR, 16), lambda: (0, 0))],
        out_specs=pl.BlockSpec((NR, 8), lambda: (0, 0)),
        out_shape=jax.ShapeDtypeStruct((NR, 8), F32),
    )


def _make_final(NV, BE=2000):
    def body(y_ref, st_ref, g_ref, b_ref, w1_ref, c1_ref, w2_ref, c2_ref,
             w3_ref, c3_ref, o_ref):
        mean = st_ref[0:1, :] / NV
        var = st_ref[1:2, :] / NV - mean * mean
        x = (g_ref[...] * (y_ref[...] - mean) * lax.rsqrt(var + 1e-5)
             + b_ref[...])
        h = jnp.maximum(jnp.dot(x, w1_ref[...], preferred_element_type=F32)
                        + c1_ref[...], 0.0)
        h = jnp.maximum(jnp.dot(h, w2_ref[...], preferred_element_type=F32)
                        + c2_ref[...], 0.0)
        o_ref[...] = (jnp.dot(h, w3_ref[...], preferred_element_type=F32)
                      + c3_ref[...])

    return pl.pallas_call(
        body,
        grid=(NV // BE,),
        in_specs=[
            pl.BlockSpec((BE, 8), lambda i: (i, 0)),
            pl.BlockSpec((2, 8), lambda i: (0, 0)),
            pl.BlockSpec((1, 8), lambda i: (0, 0)),
            pl.BlockSpec((1, 8), lambda i: (0, 0)),
            pl.BlockSpec((8, 128), lambda i: (0, 0)),
            pl.BlockSpec((1, 128), lambda i: (0, 0)),
            pl.BlockSpec((128, 128), lambda i: (0, 0)),
            pl.BlockSpec((1, 128), lambda i: (0, 0)),
            pl.BlockSpec((128, 1), lambda i: (0, 0)),
            pl.BlockSpec((1, 1), lambda i: (0, 0)),
        ],
        out_specs=pl.BlockSpec((BE, 1), lambda i: (i, 0)),
        out_shape=jax.ShapeDtypeStruct((NV, 1), F32),
    )


_g_conv0 = _make_gather(NT0, 16, 2 * E0P, 128)
_g_conv1 = _make_gather(NT1, 16, 2 * E1P, 128)
_g_convR1 = _make_gather(NT1, 32, 2 * E1P, 128)
_g_convR0 = _make_gather(NT0, 32, 2 * E0P, 128)
_g_unpool = _make_gather(NT1, 16, N0P, 80)
_s_max0 = _make_scatter_max(SN0, E0P)
_s_max1 = _make_scatter_max(SN1, E1P)
_s_pool = _make_pool(SN1, N0P, N0P // NW)
_mlp0 = _make_edge_mlp(16, E0P)
_mlp1 = _make_edge_mlp(16, E1P)
_mlpR1 = _make_edge_mlp(32, E1P)
_mlpR0 = _make_edge_mlp(32, E0P)
_red0 = _make_reduce(SN0, N0, 160)
_red1 = _make_reduce(SN1, N1, 160)
_bnp0 = _make_bn_packed(SN0 * 8 // 128, N0)
_bnp1 = _make_bn_packed(SN1 * 8 // 128, N1)
_pool_sum = _make_pool_sum(SN1 * 16 // 128)
_pool_div = _make_pool_div(2504)
_final = _make_final(N0)


def _pad_rows(a, rows):
    return jnp.concatenate(
        [a, jnp.zeros((rows - a.shape[0], a.shape[1]), a.dtype)], axis=0)


def _pad_cols(a, cols):
    return jnp.concatenate(
        [a, jnp.zeros((a.shape[0], cols - a.shape[1]), a.dtype)], axis=1)


def _pad_idx(a, n, fill):
    return jnp.concatenate(
        [a, jnp.full((n - a.shape[0],), fill, a.dtype)], axis=0)


def _expand_w(layer0, d, C):
    W, b = layer0
    K = 128 // (2 * C)
    wa = W[:d]
    wb = W[d:]
    we = jnp.zeros((128, K * 128), F32)
    for k in range(K):
        we = we.at[2 * C * k:2 * C * k + d, 128 * k:128 * (k + 1)].set(wa)
        we = we.at[2 * C * k + C:2 * C * k + C + d,
                   128 * k:128 * (k + 1)].set(wb)
    return we, b.reshape(1, 128)


def _tile128(v8):
    return jnp.tile(v8.reshape(1, 8), (1, 16))


def _conv(table, gidx, dstp, layers, d, C, EP, SN, gather, mlp, smax, red):
    we, b1 = _expand_w(layers[0], d, C)
    w2, b2 = layers[1]
    w3, b3 = layers[2]
    g = gather(table, gidx)
    xp = g.reshape(2 * EP * C // 128, 128)
    mT = mlp(xp, we, b1, w2, b2.reshape(1, 128), w3, b3.reshape(1, 8))
    parts = smax(mT, dstp)
    y, st = red(parts.reshape(NW, SN * 8 // 128, 128))
    return y, st


def kernel(x0, edge_index0, x1, edge_index1, clusters0, params):
    edge_index0 = edge_index0.astype(I32)
    edge_index1 = edge_index1.astype(I32)
    clusters0 = clusters0.astype(I32)

    src0 = _pad_idx(edge_index0[0], E0P, N0)
    dst0 = _pad_idx(edge_index0[1], E0P, N0)
    gidx0 = jnp.stack([dst0, src0], axis=1).reshape(-1)
    src1 = _pad_idx(edge_index1[0], E1P, N1)
    dst1 = _pad_idx(edge_index1[1], E1P, N1)
    gidx1 = jnp.stack([dst1, src1], axis=1).reshape(-1)

    t0 = _pad_rows(_pad_cols(x0, 16), NT0)
    y0, st0p = _conv(t0, gidx0, dst0, params["Lconv0"], 3, 16, E0P, SN0,
                     _g_conv0, _mlp0, _s_max0, _red0)
    st0 = st0p.reshape(2, 16, 8).sum(axis=1)
    h0p = _bnp0(y0, jnp.tile(st0, (1, 16)),
                _tile128(params["Lnorm0"][0]), _tile128(params["Lnorm0"][1]))
    h0 = h0p.reshape(SN0, 8)[:N0]

    h0e = jnp.concatenate(
        [h0, jnp.ones((N0, 1), F32), jnp.zeros((N0, 7), F32)], axis=1)
    h0e = _pad_rows(h0e, N0P)
    clp = _pad_idx(clusters0, N0P, N1)
    pp = _s_pool(h0e.reshape(-1), clp)
    sums = _pool_sum(pp.reshape(NW, SN1 * 16 // 128, 128))
    p1 = _pool_div(sums.reshape(SN1, 16)[:2504])[:N1]

    x1f = x1[:, :2]
    t1 = _pad_rows(_pad_cols(jnp.concatenate([x1f, p1], axis=1), 16), NT1)
    y1, st1p = _conv(t1, gidx1, dst1, params["Lconv1"], 10, 16, E1P, SN1,
                     _g_conv1, _mlp1, _s_max1, _red1)
    st1 = st1p.reshape(2, 16, 8).sum(axis=1)
    X1p = _bnp1(y1, jnp.tile(st1, (1, 16)),
                _tile128(params["Lnorm1"][0]), _tile128(params["Lnorm1"][1]))
    X1 = X1p.reshape(SN1, 8)[:N1]

    t2 = _pad_rows(_pad_cols(jnp.concatenate([x1f, p1, X1], axis=1), 32), NT1)
    y2, st2p = _conv(t2, gidx1, dst1, params["Rconv1"], 18, 32, E1P, SN1,
                     _g_convR1, _mlpR1, _s_max1, _red1)
    st2 = st2p.reshape(2, 16, 8).sum(axis=1)
    X2p = _bnp1(y2, jnp.tile(st2, (1, 16)),
                _tile128(params["Rnorm1"][0]), _tile128(params["Rnorm1"][1]))
    X2 = X2p.reshape(SN1, 8)[:N1]

    x2t = _pad_rows(_pad_cols(X2, 16), NT1)
    clg = _pad_idx(clusters0, N0P, 0)
    X3 = _g_unpool(x2t, clg)[:N0, :8]

    t3 = _pad_rows(_pad_cols(
        jnp.concatenate([x0[:, :2], h0, X3], axis=1), 32), NT0)
    y3, st3p = _conv(t3, gidx0, dst0, params["Rconv0"], 18, 32, E0P, SN0,
                     _g_convR0, _mlpR0, _s_max0, _red0)
    st3 = st3p.reshape(2, 16, 8).sum(axis=1)

    (w1, c1), (w2, c2), (w3, c3) = params["mlp_out"]
    out = _final(y3.reshape(SN0, 8)[:N0], st3,
                 params["Rnorm0"][0].reshape(1, 8),
                 params["Rnorm0"][1].reshape(1, 8),
                 w1, c1.reshape(1, 128), w2, c2.reshape(1, 128),
                 w3, c3.reshape(1, 1))
    return out


# speedup vs baseline: 3.8035x; 1.0265x over previous
import functools

import jax
import jax.numpy as jnp
from jax import lax
from jax.experimental import pallas as pl
from jax.experimental.pallas import tpu as pltpu
from jax.experimental.pallas import tpu_sc as plsc

F32 = jnp.float32
I32 = jnp.int32

N0, E0, N1, E1 = 10000, 320000, 2500, 80000
NC, NS, LANES = 2, 16, 16
NW = NC * NS

E0P = 327680
E1P = 81920
SN0 = 10240
SN1 = 2560
NT0 = N0 + 8
NT1 = N1 + 8
N0P = 10240

_MESH = plsc.VectorSubcoreMesh(core_axis_name="c", subcore_axis_name="s")
_NOTC = pltpu.CompilerParams(use_tc_tiling_on_sc=False)
_NOLAYOUT = pltpu.CompilerParams(needs_layout_passes=False)


def _take16(v, idx):
    return v.at[idx].get(mode="promise_in_bounds")


def _wid():
    return lax.axis_index("s") * NC + lax.axis_index("c")


def _make_gather(NT, C, M, GC, NB=4):
    npt = M // NW
    nch = npt // GC
    nrounds = nch // NB

    @functools.partial(
        pl.kernel,
        out_type=jax.ShapeDtypeStruct((M, C), F32),
        mesh=_MESH,
        compiler_params=_NOTC,
        scratch_types=[
            pltpu.VMEM((npt,), I32),
            pltpu.VMEM((NB, GC, C), F32),
            pltpu.SemaphoreType.DMA,
            pltpu.SemaphoreType.DMA,
        ],
    )
    def k(table_hbm, idx_hbm, out_hbm, idx_v, rows_v, sem_g, sem_o):
        base = _wid() * npt
        pltpu.sync_copy(idx_hbm.at[pl.ds(base, npt)], idx_v)

        def rnd(r, carry):
            gats = []
            for b in range(NB):
                j = r * NB + b
                gats.append(pltpu.async_copy(
                    table_hbm.at[idx_v.at[pl.ds(j * GC, GC)]],
                    rows_v.at[b], sem_g))
            outs = []
            for b in range(NB):
                j = r * NB + b
                gats[b].wait()
                outs.append(pltpu.async_copy(
                    rows_v.at[b], out_hbm.at[pl.ds(base + j * GC, GC)],
                    sem_o))
            for b in range(NB):
                outs[b].wait()
            return carry

        lax.fori_loop(0, nrounds, rnd, 0)

    return k


def _make_scatter_max(SN, EP, CH=512):
    npt = EP // NW
    nch = npt // CH
    TW = SN * 8

    @functools.partial(
        pl.kernel,
        out_type=jax.ShapeDtypeStruct((NW * TW,), F32),
        mesh=_MESH,
        compiler_params=_NOLAYOUT,
        scratch_types=[
            pltpu.VMEM((TW,), F32),
            pltpu.VMEM((2, 8, CH), F32),
            pltpu.VMEM((npt,), I32),
            pltpu.SemaphoreType.DMA,
            pltpu.SemaphoreType.DMA,
        ],
    )
    def k(m_hbm, dst_hbm, out_hbm, tbl, m_v, idx_v, sem_i, sem_m):
        wid = _wid()
        base = wid * npt
        neginf = jnp.full((LANES,), -jnp.inf, F32)

        idx_cp = pltpu.async_copy(dst_hbm.at[pl.ds(base, npt)], idx_v, sem_i)
        m_cps = [pltpu.async_copy(m_hbm.at[:, pl.ds(base, CH)],
                                  m_v.at[0], sem_m)]

        def ini(i, carry):
            tbl[pl.ds(i * LANES, LANES)] = neginf
            return carry

        lax.fori_loop(0, TW // LANES, ini, 0)
        idx_cp.wait()

        lane = lax.iota(I32, LANES)
        lane7 = lane & 7
        rot8 = lane ^ 8
        half = lane < 8

        for ch in range(nch):
            b = ch & 1
            m_cps[ch].wait()
            if ch + 1 < nch:
                off_n = base + (ch + 1) * CH
                m_cps.append(pltpu.async_copy(
                    m_hbm.at[:, pl.ds(off_n, CH)], m_v.at[1 - b], sem_m))

            def group(g, c2, ch=ch, b=b):
                dstv = idx_v[pl.ds(ch * CH + g * LANES, LANES)]
                for p in range(8):
                    patt = (lane >> 3) + 2 * p
                    dv = _take16(dstv, patt)
                    idxv = dv * 8 + lane7
                    colv = (lane >> 3) + (g * LANES + 2 * p)
                    mrow = plsc.load_gather(m_v.at[b], [lane7, colv])
                    dup = dv == _take16(dv, rot8)
                    pm = jnp.where(dup,
                                   jnp.maximum(mrow, _take16(mrow, rot8)),
                                   mrow)
                    wm = jnp.logical_or(jnp.logical_not(dup), half)
                    cur = plsc.load_gather(tbl, [idxv])
                    plsc.store_scatter(tbl, [idxv], jnp.maximum(cur, pm),
                                       mask=wm)
                return c2

            lax.fori_loop(0, CH // LANES, group, 0)

        pltpu.sync_copy(tbl, out_hbm.at[pl.ds(wid * TW, TW)])

    return k


def _make_pool(SNt, NP, CH):
    npt = NP // NW
    nch = npt // CH
    TW = SNt * 16

    @functools.partial(
        pl.kernel,
        out_type=jax.ShapeDtypeStruct((NW * TW,), F32),
        mesh=_MESH,
        compiler_params=_NOLAYOUT,
        scratch_types=[
            pltpu.VMEM((TW,), F32),
            pltpu.VMEM((CH * 16,), F32),
            pltpu.VMEM((CH,), I32),
            pltpu.SemaphoreType.DMA,
        ],
    )
    def k(val_hbm, idx_hbm, out_hbm, tbl, v_v, idx_v, sem):
        wid = _wid()
        base = wid * npt
        zero = jnp.zeros((LANES,), F32)

        def ini(i, carry):
            tbl[pl.ds(i * LANES, LANES)] = zero
            return carry

        lax.fori_loop(0, TW // LANES, ini, 0)
        lane = lax.iota(I32, LANES)

        def chunk(ch, carry):
            off = base + ch * CH
            pltpu.sync_copy(idx_hbm.at[pl.ds(off, CH)], idx_v)
            pltpu.sync_copy(val_hbm.at[pl.ds(off * 16, CH * 16)], v_v)

            def group(g, c2):
                cv = idx_v[pl.ds(g * LANES, LANES)]
                for q in range(LANES):
                    cq = _take16(cv, jnp.full((LANES,), q, I32))
                    idxv = cq * 16 + lane
                    cur = plsc.load_gather(tbl, [idxv])
                    val = v_v[pl.ds((g * LANES + q) * 16, LANES)]
                    plsc.store_scatter(tbl, [idxv], cur + val)
                return c2

            lax.fori_loop(0, CH // LANES, group, 0)
            return carry

        lax.fori_loop(0, nch, chunk, 0)
        pltpu.sync_copy(tbl, out_hbm.at[pl.ds(wid * TW, TW)])

    return k


def _make_edge_mlp(C, EP, BEp=512):
    K = 128 // (2 * C)
    MP = 2 * EP * C // 128
    KE = K * BEp

    def body(xp_ref, we_ref, b1_ref, w2_ref, b2_ref, w3_ref, b3_ref, o_ref):
        xp = xp_ref[...]
        shifted = jnp.concatenate(
            [jnp.zeros((BEp, C), F32), xp[:, :128 - C]], axis=1)
        lanes = lax.broadcasted_iota(I32, (BEp, 128), 1)
        odd = (lanes // C) % 2 == 1
        xd = xp - jnp.where(odd, shifted, 0.0)
        hp = jnp.dot(xd, we_ref[...], preferred_element_type=F32)
        h = hp.reshape(KE, 128)
        h = jnp.maximum(h + b1_ref[...], 0.0)
        h = jnp.maximum(jnp.dot(h, w2_ref[...], preferred_element_type=F32)
                        + b2_ref[...], 0.0)
        m = jnp.dot(h, w3_ref[...], preferred_element_type=F32) + b3_ref[...]
        o_ref[...] = m.T

    return pl.pallas_call(
        body,
        grid=(MP // BEp,),
        in_specs=[
            pl.BlockSpec((BEp, 128), lambda i: (i, 0)),
            pl.BlockSpec((128, K * 128), lambda i: (0, 0)),
            pl.BlockSpec((1, 128), lambda i: (0, 0)),
            pl.BlockSpec((128, 128), lambda i: (0, 0)),
            pl.BlockSpec((1, 128), lambda i: (0, 0)),
            pl.BlockSpec((128, 8), lambda i: (0, 0)),
            pl.BlockSpec((1, 8), lambda i: (0, 0)),
        ],
        out_specs=pl.BlockSpec((8, KE), lambda i: (0, i)),
        out_shape=jax.ShapeDtypeStruct((8, EP), F32),
    )


def _make_reduce(SN, NV, CHR):
    ROWS = SN * 8 // 128
    VW = NV * 8
    FR, REM = VW // 128, VW % 128

    def body(p_ref, y_ref, st_ref):
        i = pl.program_id(0)
        y = jnp.max(p_ref[...], axis=0)
        y = jnp.where(jnp.isneginf(y), 0.0, y)
        y = jnp.maximum(y, 0.0)
        y_ref[...] = y
        rows = i * CHR + lax.broadcasted_iota(I32, (CHR, 128), 0)
        valid = rows < FR
        if REM:
            lanes = lax.broadcasted_iota(I32, (CHR, 128), 1)
            valid = valid | ((rows == FR) & (lanes < REM))
        ym = jnp.where(valid, y, 0.0)
        s = jnp.sum(ym, axis=0, keepdims=True)
        ss = jnp.sum(ym * ym, axis=0, keepdims=True)

        @pl.when(i == 0)
        def _():
            st_ref[...] = jnp.zeros_like(st_ref)

        st_ref[...] += jnp.concatenate([s, ss], axis=0)

    return pl.pallas_call(
        body,
        grid=(ROWS // CHR,),
        in_specs=[pl.BlockSpec((NW, CHR, 128), lambda i: (0, i, 0))],
        out_specs=[
            pl.BlockSpec((CHR, 128), lambda i: (i, 0)),
            pl.BlockSpec((2, 128), lambda i: (0, 0)),
        ],
        out_shape=[
            jax.ShapeDtypeStruct((ROWS, 128), F32),
            jax.ShapeDtypeStruct((2, 128), F32),
        ],
    )


def _make_reduce_bn(SN, NV):
    ROWS = SN * 8 // 128
    VW = NV * 8
    FR, REM = VW // 128, VW % 128

    def body(p_ref, g_ref, b_ref, o_ref):
        y = jnp.max(p_ref[...], axis=0)
        y = jnp.where(jnp.isneginf(y), 0.0, y)
        y = jnp.maximum(y, 0.0)
        rows = lax.broadcasted_iota(I32, (ROWS, 128), 0)
        valid = rows < FR
        if REM:
            lanes = lax.broadcasted_iota(I32, (ROWS, 128), 1)
            valid = valid | ((rows == FR) & (lanes < REM))
        ym = jnp.where(valid, y, 0.0)
        s = jnp.sum(ym, axis=0, keepdims=True)
        ss = jnp.sum(ym * ym, axis=0, keepdims=True)
        sf = s[:, 0:8]
        ssf = ss[:, 0:8]
        for kk in range(1, 16):
            sf = sf + s[:, 8 * kk:8 * kk + 8]
            ssf = ssf + ss[:, 8 * kk:8 * kk + 8]
        mean = jnp.tile(sf, (1, 16)) / NV
        var = jnp.tile(ssf, (1, 16)) / NV - mean * mean
        o_ref[...] = (g_ref[...] * (y - mean) * lax.rsqrt(var + 1e-5)
                      + b_ref[...])

    return pl.pallas_call(
        body,
        in_specs=[
            pl.BlockSpec((NW, ROWS, 128), lambda: (0, 0, 0)),
            pl.BlockSpec((1, 128), lambda: (0, 0)),
            pl.BlockSpec((1, 128), lambda: (0, 0)),
        ],
        out_specs=pl.BlockSpec((ROWS, 128), lambda: (0, 0)),
        out_shape=jax.ShapeDtypeStruct((ROWS, 128), F32),
    )


def _make_bn_packed(ROWS, NV):
    def body(y_ref, st_ref, g_ref, b_ref, o_ref):
        mean = st_ref[0:1, :] / NV
        var = st_ref[1:2, :] / NV - mean * mean
        o_ref[...] = (g_ref[...] * (y_ref[...] - mean)
                      * lax.rsqrt(var + 1e-5) + b_ref[...])

    return pl.pallas_call(
        body,
        in_specs=[
            pl.BlockSpec((ROWS, 128), lambda: (0, 0)),
            pl.BlockSpec((2, 128), lambda: (0, 0)),
            pl.BlockSpec((1, 128), lambda: (0, 0)),
            pl.BlockSpec((1, 128), lambda: (0, 0)),
        ],
        out_specs=pl.BlockSpec((ROWS, 128), lambda: (0, 0)),
        out_shape=jax.ShapeDtypeStruct((ROWS, 128), F32),
    )


def _make_pool_sum(ROWS):
    def body(p_ref, o_ref):
        o_ref[...] = jnp.sum(p_ref[...], axis=0)

    return pl.pallas_call(
        body,
        in_specs=[pl.BlockSpec((NW, ROWS, 128), lambda: (0, 0, 0))],
        out_specs=pl.BlockSpec((ROWS, 128), lambda: (0, 0)),
        out_shape=jax.ShapeDtypeStruct((ROWS, 128), F32),
    )


def _make_pool_div(NR):
    def body(s_ref, o_ref):
        o_ref[...] = s_ref[:, 0:8] / jnp.maximum(s_ref[:, 8:9], 1.0)

    return pl.pallas_call(
        body,
        in_specs=[pl.BlockSpec((NR, 16), lambda: (0, 0))],
        out_specs=pl.BlockSpec((NR, 8), lambda: (0, 0)),
        out_shape=jax.ShapeDtypeStruct((NR, 8), F32),
    )


def _make_final_fused(SN, NV):
    ROWS = SN * 8 // 128
    VW = NV * 8
    FR, REM = VW // 128, VW % 128

    def body(p_ref, g_ref, b_ref, we_ref, c1_ref, w2_ref, c2_ref,
             w3_ref, c3_ref, o_ref):
        y = jnp.max(p_ref[...], axis=0)
        y = jnp.where(jnp.isneginf(y), 0.0, y)
        y = jnp.maximum(y, 0.0)
        rows = lax.broadcasted_iota(I32, (ROWS, 128), 0)
        valid = rows < FR
        if REM:
            lanes = lax.broadcasted_iota(I32, (ROWS, 128), 1)
            valid = valid | ((rows == FR) & (lanes < REM))
        ym = jnp.where(valid, y, 0.0)
        s = jnp.sum(ym, axis=0, keepdims=True)
        ss = jnp.sum(ym * ym, axis=0, keepdims=True)
        sf = s[:, 0:8]
        ssf = ss[:, 0:8]
        for kk in range(1, 16):
            sf = sf + s[:, 8 * kk:8 * kk + 8]
            ssf = ssf + ss[:, 8 * kk:8 * kk + 8]
        mean = jnp.tile(sf, (1, 16)) / NV
        var = jnp.tile(ssf, (1, 16)) / NV - mean * mean
        x = (g_ref[...] * (y - mean) * lax.rsqrt(var + 1e-5) + b_ref[...])
        hp = jnp.dot(x, we_ref[...], preferred_element_type=F32)
        h = hp.reshape(16 * ROWS, 128)
        h = jnp.maximum(h + c1_ref[...], 0.0)
        h = jnp.maximum(jnp.dot(h, w2_ref[...], preferred_element_type=F32)
                        + c2_ref[...], 0.0)
        o_ref[...] = (jnp.dot(h, w3_ref[...], preferred_element_type=F32)
                      + c3_ref[...])

    return pl.pallas_call(
        body,
        in_specs=[
            pl.BlockSpec((NW, ROWS, 128), lambda: (0, 0, 0)),
            pl.BlockSpec((1, 128), lambda: (0, 0)),
            pl.BlockSpec((1, 128), lambda: (0, 0)),
            pl.BlockSpec((128, 2048), lambda: (0, 0)),
            pl.BlockSpec((1, 128), lambda: (0, 0)),
            pl.BlockSpec((128, 128), lambda: (0, 0)),
            pl.BlockSpec((1, 128), lambda: (0, 0)),
            pl.BlockSpec((128, 1), lambda: (0, 0)),
            pl.BlockSpec((1, 1), lambda: (0, 0)),
        ],
        out_specs=pl.BlockSpec((16 * ROWS, 1), lambda: (0, 0)),
        out_shape=jax.ShapeDtypeStruct((16 * ROWS, 1), F32),
    )


def _make_final(NV, BE=2000):
    def body(y_ref, st_ref, g_ref, b_ref, w1_ref, c1_ref, w2_ref, c2_ref,
             w3_ref, c3_ref, o_ref):
        mean = st_ref[0:1, :] / NV
        var = st_ref[1:2, :] / NV - mean * mean
        x = (g_ref[...] * (y_ref[...] - mean) * lax.rsqrt(var + 1e-5)
             + b_ref[...])
        h = jnp.maximum(jnp.dot(x, w1_ref[...], preferred_element_type=F32)
                        + c1_ref[...], 0.0)
        h = jnp.maximum(jnp.dot(h, w2_ref[...], preferred_element_type=F32)
                        + c2_ref[...], 0.0)
        o_ref[...] = (jnp.dot(h, w3_ref[...], preferred_element_type=F32)
                      + c3_ref[...])

    return pl.pallas_call(
        body,
        grid=(NV // BE,),
        in_specs=[
            pl.BlockSpec((BE, 8), lambda i: (i, 0)),
            pl.BlockSpec((2, 8), lambda i: (0, 0)),
            pl.BlockSpec((1, 8), lambda i: (0, 0)),
            pl.BlockSpec((1, 8), lambda i: (0, 0)),
            pl.BlockSpec((8, 128), lambda i: (0, 0)),
            pl.BlockSpec((1, 128), lambda i: (0, 0)),
            pl.BlockSpec((128, 128), lambda i: (0, 0)),
            pl.BlockSpec((1, 128), lambda i: (0, 0)),
            pl.BlockSpec((128, 1), lambda i: (0, 0)),
            pl.BlockSpec((1, 1), lambda i: (0, 0)),
        ],
        out_specs=pl.BlockSpec((BE, 1), lambda i: (i, 0)),
        out_shape=jax.ShapeDtypeStruct((NV, 1), F32),
    )


_g_conv0 = _make_gather(NT0, 16, 2 * E0P, 128)
_g_conv1 = _make_gather(NT1, 16, 2 * E1P, 128)
_g_convR1 = _make_gather(NT1, 32, 2 * E1P, 128)
_g_convR0 = _make_gather(NT0, 32, 2 * E0P, 128)
_g_unpool = _make_gather(NT1, 16, N0P, 80)
_s_max0 = _make_scatter_max(SN0, E0P)
_s_max1 = _make_scatter_max(SN1, E1P)
_s_pool = _make_pool(SN1, N0P, N0P // NW)
_mlp0 = _make_edge_mlp(16, E0P)
_mlp1 = _make_edge_mlp(16, E1P)
_mlpR1 = _make_edge_mlp(32, E1P)
_mlpR0 = _make_edge_mlp(32, E0P)
_redbn0 = _make_reduce_bn(SN0, N0)
_redbn1 = _make_reduce_bn(SN1, N1)
_finalf = _make_final_fused(SN0, N0)
_pool_sum = _make_pool_sum(SN1 * 16 // 128)
_pool_div = _make_pool_div(2504)
_final = _make_final(N0)


def _pad_rows(a, rows):
    return jnp.concatenate(
        [a, jnp.zeros((rows - a.shape[0], a.shape[1]), a.dtype)], axis=0)


def _pad_cols(a, cols):
    return jnp.concatenate(
        [a, jnp.zeros((a.shape[0], cols - a.shape[1]), a.dtype)], axis=1)


def _pad_idx(a, n, fill):
    return jnp.concatenate(
        [a, jnp.full((n - a.shape[0],), fill, a.dtype)], axis=0)


def _expand_w(layer0, d, C):
    W, b = layer0
    K = 128 // (2 * C)
    wa = W[:d]
    wb = W[d:]
    we = jnp.zeros((128, K * 128), F32)
    for k in range(K):
        we = we.at[2 * C * k:2 * C * k + d, 128 * k:128 * (k + 1)].set(wa)
        we = we.at[2 * C * k + C:2 * C * k + C + d,
                   128 * k:128 * (k + 1)].set(wb)
    return we, b.reshape(1, 128)


def _tile128(v8):
    return jnp.tile(v8.reshape(1, 8), (1, 16))


def _conv(table, gidx, dstp, layers, d, C, EP, SN, gather, mlp, smax):
    we, b1 = _expand_w(layers[0], d, C)
    w2, b2 = layers[1]
    w3, b3 = layers[2]
    g = gather(table, gidx)
    xp = g.reshape(2 * EP * C // 128, 128)
    mT = mlp(xp, we, b1, w2, b2.reshape(1, 128), w3, b3.reshape(1, 8))
    parts = smax(mT, dstp)
    return parts.reshape(NW, SN * 8 // 128, 128)


def kernel(x0, edge_index0, x1, edge_index1, clusters0, params):
    edge_index0 = edge_index0.astype(I32)
    edge_index1 = edge_index1.astype(I32)
    clusters0 = clusters0.astype(I32)

    src0 = _pad_idx(edge_index0[0], E0P, N0)
    dst0 = _pad_idx(edge_index0[1], E0P, N0)
    gidx0 = jnp.stack([dst0, src0], axis=1).reshape(-1)
    src1 = _pad_idx(edge_index1[0], E1P, N1)
    dst1 = _pad_idx(edge_index1[1], E1P, N1)
    gidx1 = jnp.stack([dst1, src1], axis=1).reshape(-1)

    t0 = _pad_rows(_pad_cols(x0, 16), NT0)
    parts0 = _conv(t0, gidx0, dst0, params["Lconv0"], 3, 16, E0P, SN0,
                   _g_conv0, _mlp0, _s_max0)
    h0p = _redbn0(parts0, _tile128(params["Lnorm0"][0]),
                  _tile128(params["Lnorm0"][1]))
    h0 = h0p.reshape(SN0, 8)[:N0]

    h0e = jnp.concatenate(
        [h0, jnp.ones((N0, 1), F32), jnp.zeros((N0, 7), F32)], axis=1)
    h0e = _pad_rows(h0e, N0P)
    clp = _pad_idx(clusters0, N0P, N1)
    pp = _s_pool(h0e.reshape(-1), clp)
    sums = _pool_sum(pp.reshape(NW, SN1 * 16 // 128, 128))
    p1 = _pool_div(sums.reshape(SN1, 16)[:2504])[:N1]

    x1f = x1[:, :2]
    t1 = _pad_rows(_pad_cols(jnp.concatenate([x1f, p1], axis=1), 16), NT1)
    parts1 = _conv(t1, gidx1, dst1, params["Lconv1"], 10, 16, E1P, SN1,
                   _g_conv1, _mlp1, _s_max1)
    X1p = _redbn1(parts1, _tile128(params["Lnorm1"][0]),
                  _tile128(params["Lnorm1"][1]))
    X1 = X1p.reshape(SN1, 8)[:N1]

    t2 = _pad_rows(_pad_cols(jnp.concatenate([x1f, p1, X1], axis=1), 32), NT1)
    parts2 = _conv(t2, gidx1, dst1, params["Rconv1"], 18, 32, E1P, SN1,
                   _g_convR1, _mlpR1, _s_max1)
    X2p = _redbn1(parts2, _tile128(params["Rnorm1"][0]),
                  _tile128(params["Rnorm1"][1]))
    X2 = X2p.reshape(SN1, 8)[:N1]

    x2t = _pad_rows(_pad_cols(X2, 16), NT1)
    clg = _pad_idx(clusters0, N0P, 0)
    X3 = _g_unpool(x2t, clg)[:N0, :8]

    t3 = _pad_rows(_pad_cols(
        jnp.concatenate([x0[:, :2], h0, X3], axis=1), 32), NT0)
    parts3 = _conv(t3, gidx0, dst0, params["Rconv0"], 18, 32, E0P, SN0,
                   _g_convR0, _mlpR0, _s_max0)

    (w1, c1), (w2, c2), (w3, c3) = params["mlp_out"]
    we_f = jnp.zeros((128, 2048), F32)
    for k in range(16):
        we_f = we_f.at[8 * k:8 * k + 8, 128 * k:128 * (k + 1)].set(w1)
    out = _finalf(parts3, _tile128(params["Rnorm0"][0]),
                  _tile128(params["Rnorm0"][1]),
                  we_f, c1.reshape(1, 128), w2, c2.reshape(1, 128),
                  w3, c3.reshape(1, 1))
    return out[:N0]


# speedup vs baseline: 3.8642x; 1.0160x over previous
import functools

import jax
import jax.numpy as jnp
from jax import lax
from jax.experimental import pallas as pl
from jax.experimental.pallas import tpu as pltpu
from jax.experimental.pallas import tpu_sc as plsc

F32 = jnp.float32
I32 = jnp.int32

N0, E0, N1, E1 = 10000, 320000, 2500, 80000
NC, NS, LANES = 2, 16, 16
NW = NC * NS

E0P = 327680
E1P = 81920
SN0 = 10240
SN1 = 2560
NT0 = N0 + 8
NT1 = N1 + 8
N0P = 10240

_MESH = plsc.VectorSubcoreMesh(core_axis_name="c", subcore_axis_name="s")
_NOTC = pltpu.CompilerParams(use_tc_tiling_on_sc=False)
_NOLAYOUT = pltpu.CompilerParams(needs_layout_passes=False)


def _take16(v, idx):
    return v.at[idx].get(mode="promise_in_bounds")


def _wid():
    return lax.axis_index("s") * NC + lax.axis_index("c")


def _make_gather(NT, C, M, GC, NB=8):
    npt = M // NW
    nch = npt // GC
    nrounds = nch // NB

    @functools.partial(
        pl.kernel,
        out_type=jax.ShapeDtypeStruct((M, C), F32),
        mesh=_MESH,
        compiler_params=_NOTC,
        scratch_types=[
            pltpu.VMEM((npt,), I32),
            pltpu.VMEM((NB, GC, C), F32),
            pltpu.SemaphoreType.DMA,
            pltpu.SemaphoreType.DMA,
        ],
    )
    def k(table_hbm, idx_hbm, out_hbm, idx_v, rows_v, sem_g, sem_o):
        base = _wid() * npt
        pltpu.sync_copy(idx_hbm.at[pl.ds(base, npt)], idx_v)

        def rnd(r, carry):
            gats = []
            for b in range(NB):
                j = r * NB + b
                gats.append(pltpu.async_copy(
                    table_hbm.at[idx_v.at[pl.ds(j * GC, GC)]],
                    rows_v.at[b], sem_g))
            outs = []
            for b in range(NB):
                j = r * NB + b
                gats[b].wait()
                outs.append(pltpu.async_copy(
                    rows_v.at[b], out_hbm.at[pl.ds(base + j * GC, GC)],
                    sem_o))
            for b in range(NB):
                outs[b].wait()
            return carry

        lax.fori_loop(0, nrounds, rnd, 0)

    return k


def _make_scatter_max(SN, EP, CH=512):
    npt = EP // NW
    nch = npt // CH
    TW = SN * 8

    @functools.partial(
        pl.kernel,
        out_type=jax.ShapeDtypeStruct((NW * TW,), F32),
        mesh=_MESH,
        compiler_params=_NOLAYOUT,
        scratch_types=[
            pltpu.VMEM((TW,), F32),
            pltpu.VMEM((2, 8, CH), F32),
            pltpu.VMEM((npt,), I32),
            pltpu.SemaphoreType.DMA,
            pltpu.SemaphoreType.DMA,
        ],
    )
    def k(m_hbm, dst_hbm, out_hbm, tbl, m_v, idx_v, sem_i, sem_m):
        wid = _wid()
        base = wid * npt
        neginf = jnp.full((LANES,), -jnp.inf, F32)

        idx_cp = pltpu.async_copy(dst_hbm.at[pl.ds(base, npt)], idx_v, sem_i)
        m_cps = [pltpu.async_copy(m_hbm.at[:, pl.ds(base, CH)],
                                  m_v.at[0], sem_m)]

        def ini(i, carry):
            tbl[pl.ds(i * LANES, LANES)] = neginf
            return carry

        lax.fori_loop(0, TW // LANES, ini, 0)
        idx_cp.wait()

        lane = lax.iota(I32, LANES)
        lane7 = lane & 7
        rot8 = lane ^ 8
        half = lane < 8

        for ch in range(nch):
            b = ch & 1
            m_cps[ch].wait()
            if ch + 1 < nch:
                off_n = base + (ch + 1) * CH
                m_cps.append(pltpu.async_copy(
                    m_hbm.at[:, pl.ds(off_n, CH)], m_v.at[1 - b], sem_m))

            def group(g, c2, ch=ch, b=b):
                dstv = idx_v[pl.ds(ch * CH + g * LANES, LANES)]
                for p in range(8):
                    patt = (lane >> 3) + 2 * p
                    dv = _take16(dstv, patt)
                    idxv = dv * 8 + lane7
                    colv = (lane >> 3) + (g * LANES + 2 * p)
                    mrow = plsc.load_gather(m_v.at[b], [lane7, colv])
                    dup = dv == _take16(dv, rot8)
                    pm = jnp.where(dup,
                                   jnp.maximum(mrow, _take16(mrow, rot8)),
                                   mrow)
                    wm = jnp.logical_or(jnp.logical_not(dup), half)
                    cur = plsc.load_gather(tbl, [idxv])
                    plsc.store_scatter(tbl, [idxv], jnp.maximum(cur, pm),
                                       mask=wm)
                return c2

            lax.fori_loop(0, CH // LANES, group, 0)

        pltpu.sync_copy(tbl, out_hbm.at[pl.ds(wid * TW, TW)])

    return k


def _make_pool(SNt, NP, CH):
    npt = NP // NW
    nch = npt // CH
    TW = SNt * 16

    @functools.partial(
        pl.kernel,
        out_type=jax.ShapeDtypeStruct((NW * TW,), F32),
        mesh=_MESH,
        compiler_params=_NOLAYOUT,
        scratch_types=[
            pltpu.VMEM((TW,), F32),
            pltpu.VMEM((CH * 16,), F32),
            pltpu.VMEM((CH,), I32),
            pltpu.SemaphoreType.DMA,
        ],
    )
    def k(val_hbm, idx_hbm, out_hbm, tbl, v_v, idx_v, sem):
        wid = _wid()
        base = wid * npt
        zero = jnp.zeros((LANES,), F32)

        def ini(i, carry):
            tbl[pl.ds(i * LANES, LANES)] = zero
            return carry

        lax.fori_loop(0, TW // LANES, ini, 0)
        lane = lax.iota(I32, LANES)

        def chunk(ch, carry):
            off = base + ch * CH
            pltpu.sync_copy(idx_hbm.at[pl.ds(off, CH)], idx_v)
            pltpu.sync_copy(val_hbm.at[pl.ds(off * 16, CH * 16)], v_v)

            def group(g, c2):
                cv = idx_v[pl.ds(g * LANES, LANES)]
                for q in range(LANES):
                    cq = _take16(cv, jnp.full((LANES,), q, I32))
                    idxv = cq * 16 + lane
                    cur = plsc.load_gather(tbl, [idxv])
                    val = v_v[pl.ds((g * LANES + q) * 16, LANES)]
                    plsc.store_scatter(tbl, [idxv], cur + val)
                return c2

            lax.fori_loop(0, CH // LANES, group, 0)
            return carry

        lax.fori_loop(0, nch, chunk, 0)
        pltpu.sync_copy(tbl, out_hbm.at[pl.ds(wid * TW, TW)])

    return k


def _make_edge_mlp(C, EP, BEp=512):
    K = 128 // (2 * C)
    MP = 2 * EP * C // 128
    KE = K * BEp

    def body(xp_ref, we_ref, b1_ref, w2_ref, b2_ref, w3_ref, b3_ref, o_ref):
        xp = xp_ref[...]
        shifted = jnp.concatenate(
            [jnp.zeros((BEp, C), F32), xp[:, :128 - C]], axis=1)
        lanes = lax.broadcasted_iota(I32, (BEp, 128), 1)
        odd = (lanes // C) % 2 == 1
        xd = xp - jnp.where(odd, shifted, 0.0)
        hp = jnp.dot(xd, we_ref[...], preferred_element_type=F32)
        h = hp.reshape(KE, 128)
        h = jnp.maximum(h + b1_ref[...], 0.0)
        h = jnp.maximum(jnp.dot(h, w2_ref[...], preferred_element_type=F32)
                        + b2_ref[...], 0.0)
        m = jnp.dot(h, w3_ref[...], preferred_element_type=F32) + b3_ref[...]
        o_ref[...] = m.T

    return pl.pallas_call(
        body,
        grid=(MP // BEp,),
        in_specs=[
            pl.BlockSpec((BEp, 128), lambda i: (i, 0)),
            pl.BlockSpec((128, K * 128), lambda i: (0, 0)),
            pl.BlockSpec((1, 128), lambda i: (0, 0)),
            pl.BlockSpec((128, 128), lambda i: (0, 0)),
            pl.BlockSpec((1, 128), lambda i: (0, 0)),
            pl.BlockSpec((128, 8), lambda i: (0, 0)),
            pl.BlockSpec((1, 8), lambda i: (0, 0)),
        ],
        out_specs=pl.BlockSpec((8, KE), lambda i: (0, i)),
        out_shape=jax.ShapeDtypeStruct((8, EP), F32),
    )


def _make_reduce(SN, NV, CHR):
    ROWS = SN * 8 // 128
    VW = NV * 8
    FR, REM = VW // 128, VW % 128

    def body(p_ref, y_ref, st_ref):
        i = pl.program_id(0)
        y = jnp.max(p_ref[...], axis=0)
        y = jnp.where(jnp.isneginf(y), 0.0, y)
        y = jnp.maximum(y, 0.0)
        y_ref[...] = y
        rows = i * CHR + lax.broadcasted_iota(I32, (CHR, 128), 0)
        valid = rows < FR
        if REM:
            lanes = lax.broadcasted_iota(I32, (CHR, 128), 1)
            valid = valid | ((rows == FR) & (lanes < REM))
        ym = jnp.where(valid, y, 0.0)
        s = jnp.sum(ym, axis=0, keepdims=True)
        ss = jnp.sum(ym * ym, axis=0, keepdims=True)

        @pl.when(i == 0)
        def _():
            st_ref[...] = jnp.zeros_like(st_ref)

        st_ref[...] += jnp.concatenate([s, ss], axis=0)

    return pl.pallas_call(
        body,
        grid=(ROWS // CHR,),
        in_specs=[pl.BlockSpec((NW, CHR, 128), lambda i: (0, i, 0))],
        out_specs=[
            pl.BlockSpec((CHR, 128), lambda i: (i, 0)),
            pl.BlockSpec((2, 128), lambda i: (0, 0)),
        ],
        out_shape=[
            jax.ShapeDtypeStruct((ROWS, 128), F32),
            jax.ShapeDtypeStruct((2, 128), F32),
        ],
    )


def _make_reduce_bn(SN, NV):
    ROWS = SN * 8 // 128
    VW = NV * 8
    FR, REM = VW // 128, VW % 128

    def body(p_ref, g_ref, b_ref, o_ref):
        y = jnp.max(p_ref[...], axis=0)
        y = jnp.where(jnp.isneginf(y), 0.0, y)
        y = jnp.maximum(y, 0.0)
        rows = lax.broadcasted_iota(I32, (ROWS, 128), 0)
        valid = rows < FR
        if REM:
            lanes = lax.broadcasted_iota(I32, (ROWS, 128), 1)
            valid = valid | ((rows == FR) & (lanes < REM))
        ym = jnp.where(valid, y, 0.0)
        s = jnp.sum(ym, axis=0, keepdims=True)
        ss = jnp.sum(ym * ym, axis=0, keepdims=True)
        sf = s[:, 0:8]
        ssf = ss[:, 0:8]
        for kk in range(1, 16):
            sf = sf + s[:, 8 * kk:8 * kk + 8]
            ssf = ssf + ss[:, 8 * kk:8 * kk + 8]
        mean = jnp.tile(sf, (1, 16)) / NV
        var = jnp.tile(ssf, (1, 16)) / NV - mean * mean
        o_ref[...] = (g_ref[...] * (y - mean) * lax.rsqrt(var + 1e-5)
                      + b_ref[...])

    return pl.pallas_call(
        body,
        in_specs=[
            pl.BlockSpec((NW, ROWS, 128), lambda: (0, 0, 0)),
            pl.BlockSpec((1, 128), lambda: (0, 0)),
            pl.BlockSpec((1, 128), lambda: (0, 0)),
        ],
        out_specs=pl.BlockSpec((ROWS, 128), lambda: (0, 0)),
        out_shape=jax.ShapeDtypeStruct((ROWS, 128), F32),
    )


def _make_bn_packed(ROWS, NV):
    def body(y_ref, st_ref, g_ref, b_ref, o_ref):
        mean = st_ref[0:1, :] / NV
        var = st_ref[1:2, :] / NV - mean * mean
        o_ref[...] = (g_ref[...] * (y_ref[...] - mean)
                      * lax.rsqrt(var + 1e-5) + b_ref[...])

    return pl.pallas_call(
        body,
        in_specs=[
            pl.BlockSpec((ROWS, 128), lambda: (0, 0)),
            pl.BlockSpec((2, 128), lambda: (0, 0)),
            pl.BlockSpec((1, 128), lambda: (0, 0)),
            pl.BlockSpec((1, 128), lambda: (0, 0)),
        ],
        out_specs=pl.BlockSpec((ROWS, 128), lambda: (0, 0)),
        out_shape=jax.ShapeDtypeStruct((ROWS, 128), F32),
    )


def _make_pool_sum(ROWS):
    def body(p_ref, o_ref):
        o_ref[...] = jnp.sum(p_ref[...], axis=0)

    return pl.pallas_call(
        body,
        in_specs=[pl.BlockSpec((NW, ROWS, 128), lambda: (0, 0, 0))],
        out_specs=pl.BlockSpec((ROWS, 128), lambda: (0, 0)),
        out_shape=jax.ShapeDtypeStruct((ROWS, 128), F32),
    )


def _make_pool_div(NR):
    def body(s_ref, o_ref):
        o_ref[...] = s_ref[:, 0:8] / jnp.maximum(s_ref[:, 8:9], 1.0)

    return pl.pallas_call(
        body,
        in_specs=[pl.BlockSpec((NR, 16), lambda: (0, 0))],
        out_specs=pl.BlockSpec((NR, 8), lambda: (0, 0)),
        out_shape=jax.ShapeDtypeStruct((NR, 8), F32),
    )


def _make_final_fused(SN, NV):
    ROWS = SN * 8 // 128
    VW = NV * 8
    FR, REM = VW // 128, VW % 128

    def body(p_ref, g_ref, b_ref, we_ref, c1_ref, w2_ref, c2_ref,
             w3_ref, c3_ref, o_ref):
        y = jnp.max(p_ref[...], axis=0)
        y = jnp.where(jnp.isneginf(y), 0.0, y)
        y = jnp.maximum(y, 0.0)
        rows = lax.broadcasted_iota(I32, (ROWS, 128), 0)
        valid = rows < FR
        if REM:
            lanes = lax.broadcasted_iota(I32, (ROWS, 128), 1)
            valid = valid | ((rows == FR) & (lanes < REM))
        ym = jnp.where(valid, y, 0.0)
        s = jnp.sum(ym, axis=0, keepdims=True)
        ss = jnp.sum(ym * ym, axis=0, keepdims=True)
        sf = s[:, 0:8]
        ssf = ss[:, 0:8]
        for kk in range(1, 16):
            sf = sf + s[:, 8 * kk:8 * kk + 8]
            ssf = ssf + ss[:, 8 * kk:8 * kk + 8]
        mean = jnp.tile(sf, (1, 16)) / NV
        var = jnp.tile(ssf, (1, 16)) / NV - mean * mean
        x = (g_ref[...] * (y - mean) * lax.rsqrt(var + 1e-5) + b_ref[...])
        hp = jnp.dot(x, we_ref[...], preferred_element_type=F32)
        h = hp.reshape(16 * ROWS, 128)
        h = jnp.maximum(h + c1_ref[...], 0.0)
        h = jnp.maximum(jnp.dot(h, w2_ref[...], preferred_element_type=F32)
                        + c2_ref[...], 0.0)
        o_ref[...] = (jnp.dot(h, w3_ref[...], preferred_element_type=F32)
                      + c3_ref[...])

    return pl.pallas_call(
        body,
        in_specs=[
            pl.BlockSpec((NW, ROWS, 128), lambda: (0, 0, 0)),
            pl.BlockSpec((1, 128), lambda: (0, 0)),
            pl.BlockSpec((1, 128), lambda: (0, 0)),
            pl.BlockSpec((128, 2048), lambda: (0, 0)),
            pl.BlockSpec((1, 128), lambda: (0, 0)),
            pl.BlockSpec((128, 128), lambda: (0, 0)),
            pl.BlockSpec((1, 128), lambda: (0, 0)),
            pl.BlockSpec((128, 1), lambda: (0, 0)),
            pl.BlockSpec((1, 1), lambda: (0, 0)),
        ],
        out_specs=pl.BlockSpec((16 * ROWS, 1), lambda: (0, 0)),
        out_shape=jax.ShapeDtypeStruct((16 * ROWS, 1), F32),
    )


def _make_final(NV, BE=2000):
    def body(y_ref, st_ref, g_ref, b_ref, w1_ref, c1_ref, w2_ref, c2_ref,
             w3_ref, c3_ref, o_ref):
        mean = st_ref[0:1, :] / NV
        var = st_ref[1:2, :] / NV - mean * mean
        x = (g_ref[...] * (y_ref[...] - mean) * lax.rsqrt(var + 1e-5)
             + b_ref[...])
        h = jnp.maximum(jnp.dot(x, w1_ref[...], preferred_element_type=F32)
                        + c1_ref[...], 0.0)
        h = jnp.maximum(jnp.dot(h, w2_ref[...], preferred_element_type=F32)
                        + c2_ref[...], 0.0)
        o_ref[...] = (jnp.dot(h, w3_ref[...], preferred_element_type=F32)
                      + c3_ref[...])

    return pl.pallas_call(
        body,
        grid=(NV // BE,),
        in_specs=[
            pl.BlockSpec((BE, 8), lambda i: (i, 0)),
            pl.BlockSpec((2, 8), lambda i: (0, 0)),
            pl.BlockSpec((1, 8), lambda i: (0, 0)),
            pl.BlockSpec((1, 8), lambda i: (0, 0)),
            pl.BlockSpec((8, 128), lambda i: (0, 0)),
            pl.BlockSpec((1, 128), lambda i: (0, 0)),
            pl.BlockSpec((128, 128), lambda i: (0, 0)),
            pl.BlockSpec((1, 128), lambda i: (0, 0)),
            pl.BlockSpec((128, 1), lambda i: (0, 0)),
            pl.BlockSpec((1, 1), lambda i: (0, 0)),
        ],
        out_specs=pl.BlockSpec((BE, 1), lambda i: (i, 0)),
        out_shape=jax.ShapeDtypeStruct((NV, 1), F32),
    )


_g_conv0 = _make_gather(NT0, 16, 2 * E0P, 128)
_g_conv1 = _make_gather(NT1, 16, 2 * E1P, 128)
_g_convR1 = _make_gather(NT1, 32, 2 * E1P, 128)
_g_convR0 = _make_gather(NT0, 32, 2 * E0P, 128)
_g_unpool = _make_gather(NT1, 16, N0P, 80, NB=4)
_s_max0 = _make_scatter_max(SN0, E0P)
_s_max1 = _make_scatter_max(SN1, E1P)
_s_pool = _make_pool(SN1, N0P, N0P // NW)
_mlp0 = _make_edge_mlp(16, E0P)
_mlp1 = _make_edge_mlp(16, E1P)
_mlpR1 = _make_edge_mlp(32, E1P)
_mlpR0 = _make_edge_mlp(32, E0P)
_redbn0 = _make_reduce_bn(SN0, N0)
_redbn1 = _make_reduce_bn(SN1, N1)
_finalf = _make_final_fused(SN0, N0)
_pool_sum = _make_pool_sum(SN1 * 16 // 128)
_pool_div = _make_pool_div(2504)
_final = _make_final(N0)


def _pad_rows(a, rows):
    return jnp.concatenate(
        [a, jnp.zeros((rows - a.shape[0], a.shape[1]), a.dtype)], axis=0)


def _pad_cols(a, cols):
    return jnp.concatenate(
        [a, jnp.zeros((a.shape[0], cols - a.shape[1]), a.dtype)], axis=1)


def _pad_idx(a, n, fill):
    return jnp.concatenate(
        [a, jnp.full((n - a.shape[0],), fill, a.dtype)], axis=0)


def _expand_w(layer0, d, C):
    W, b = layer0
    K = 128 // (2 * C)
    wa = W[:d]
    wb = W[d:]
    we = jnp.zeros((128, K * 128), F32)
    for k in range(K):
        we = we.at[2 * C * k:2 * C * k + d, 128 * k:128 * (k + 1)].set(wa)
        we = we.at[2 * C * k + C:2 * C * k + C + d,
                   128 * k:128 * (k + 1)].set(wb)
    return we, b.reshape(1, 128)


def _tile128(v8):
    return jnp.tile(v8.reshape(1, 8), (1, 16))


def _conv(table, gidx, dstp, layers, d, C, EP, SN, gather, mlp, smax):
    we, b1 = _expand_w(layers[0], d, C)
    w2, b2 = layers[1]
    w3, b3 = layers[2]
    g = gather(table, gidx)
    xp = g.reshape(2 * EP * C // 128, 128)
    mT = mlp(xp, we, b1, w2, b2.reshape(1, 128), w3, b3.reshape(1, 8))
    parts = smax(mT, dstp)
    return parts.reshape(NW, SN * 8 // 128, 128)


def kernel(x0, edge_index0, x1, edge_index1, clusters0, params):
    edge_index0 = edge_index0.astype(I32)
    edge_index1 = edge_index1.astype(I32)
    clusters0 = clusters0.astype(I32)

    src0 = _pad_idx(edge_index0[0], E0P, N0)
    dst0 = _pad_idx(edge_index0[1], E0P, N0)
    gidx0 = jnp.stack([dst0, src0], axis=1).reshape(-1)
    src1 = _pad_idx(edge_index1[0], E1P, N1)
    dst1 = _pad_idx(edge_index1[1], E1P, N1)
    gidx1 = jnp.stack([dst1, src1], axis=1).reshape(-1)

    t0 = _pad_rows(_pad_cols(x0, 16), NT0)
    parts0 = _conv(t0, gidx0, dst0, params["Lconv0"], 3, 16, E0P, SN0,
                   _g_conv0, _mlp0, _s_max0)
    h0p = _redbn0(parts0, _tile128(params["Lnorm0"][0]),
                  _tile128(params["Lnorm0"][1]))
    h0 = h0p.reshape(SN0, 8)[:N0]

    h0e = jnp.concatenate(
        [h0, jnp.ones((N0, 1), F32), jnp.zeros((N0, 7), F32)], axis=1)
    h0e = _pad_rows(h0e, N0P)
    clp = _pad_idx(clusters0, N0P, N1)
    pp = _s_pool(h0e.reshape(-1), clp)
    sums = _pool_sum(pp.reshape(NW, SN1 * 16 // 128, 128))
    p1 = _pool_div(sums.reshape(SN1, 16)[:2504])[:N1]

    x1f = x1[:, :2]
    t1 = _pad_rows(_pad_cols(jnp.concatenate([x1f, p1], axis=1), 16), NT1)
    parts1 = _conv(t1, gidx1, dst1, params["Lconv1"], 10, 16, E1P, SN1,
                   _g_conv1, _mlp1, _s_max1)
    X1p = _redbn1(parts1, _tile128(params["Lnorm1"][0]),
                  _tile128(params["Lnorm1"][1]))
    X1 = X1p.reshape(SN1, 8)[:N1]

    t2 = _pad_rows(_pad_cols(jnp.concatenate([x1f, p1, X1], axis=1), 32), NT1)
    parts2 = _conv(t2, gidx1, dst1, params["Rconv1"], 18, 32, E1P, SN1,
                   _g_convR1, _mlpR1, _s_max1)
    X2p = _redbn1(parts2, _tile128(params["Rnorm1"][0]),
                  _tile128(params["Rnorm1"][1]))
    X2 = X2p.reshape(SN1, 8)[:N1]

    x2t = _pad_rows(_pad_cols(X2, 16), NT1)
    clg = _pad_idx(clusters0, N0P, 0)
    X3 = _g_unpool(x2t, clg)[:N0, :8]

    t3 = _pad_rows(_pad_cols(
        jnp.concatenate([x0[:, :2], h0, X3], axis=1), 32), NT0)
    parts3 = _conv(t3, gidx0, dst0, params["Rconv0"], 18, 32, E0P, SN0,
                   _g_convR0, _mlpR0, _s_max0)

    (w1, c1), (w2, c2), (w3, c3) = params["mlp_out"]
    we_f = jnp.zeros((128, 2048), F32)
    for k in range(16):
        we_f = we_f.at[8 * k:8 * k + 8, 128 * k:128 * (k + 1)].set(w1)
    out = _finalf(parts3, _tile128(params["Rnorm0"][0]),
                  _tile128(params["Rnorm0"][1]),
                  we_f, c1.reshape(1, 128), w2, c2.reshape(1, 128),
                  w3, c3.reshape(1, 1))
    return out[:N0]
